# baseline (speedup 1.0000x reference)
import functools

import jax
import jax.numpy as jnp
from jax import lax
from jax.experimental import pallas as pl
from jax.experimental.pallas import tpu as pltpu
from jax.experimental.pallas import tpu_sc as plsc

N = 10000
E = 320000
NFEAT = 128
NHID = 128
NCLASS = 64

NC = 2
NS = 16
NW = NC * NS
EPW = E // NW
K = 64
EPW_P = 10240
C = EPW_P // K
N_PAD = 10240
ROWS_PER_TILE = N_PAD // NS
ZR = 8
SHIFT = 14


def _make_spmm(F):
    mesh = plsc.VectorSubcoreMesh(core_axis_name="c", subcore_axis_name="s")

    @functools.partial(
        pl.kernel,
        mesh=mesh,
        out_type=jax.ShapeDtypeStruct((NC, N_PAD, F), jnp.float32),
        compiler_params=pltpu.CompilerParams(use_tc_tiling_on_sc=False),
        scratch_types=[
            pltpu.VMEM_SHARED((N_PAD, F), jnp.float32),
            pltpu.VMEM((C, K), jnp.int32),
            pltpu.VMEM((EPW_P,), jnp.float32),
            pltpu.VMEM((1, K), jnp.int32),
            pltpu.VMEM((1, K), jnp.int32),
            pltpu.VMEM((K, F), jnp.float32),
            pltpu.VMEM((ZR, F), jnp.float32),
            pltpu.SemaphoreType.DMA,
        ],
    )
    def spmm(support_hbm, pk_hbm, w_hbm, out_hbm,
             acc, pk_v, w_v, sbuf, dbuf, rows_v, zeros_v, zsem):
        cid = lax.axis_index("c")
        sid = lax.axis_index("s")
        wid = sid * NC + cid

        for r in range(ZR):
            for j in range(F // 16):
                zeros_v[r, pl.ds(j * 16, 16)] = jnp.zeros((16,), jnp.float32)
        base = sid * ROWS_PER_TILE
        NZ = ROWS_PER_TILE // ZR

        def zissue(i, _):
            pltpu.async_copy(zeros_v, acc.at[pl.ds(base + i * ZR, ZR)], zsem)
            return 0
        lax.fori_loop(0, NZ, zissue, 0)

        def zdrain(i, _):
            pltpu.make_async_copy(zeros_v, acc.at[pl.ds(base, ZR)], zsem).wait()
            return 0
        lax.fori_loop(0, NZ, zdrain, 0)
        plsc.subcore_barrier()

        pltpu.sync_copy(pk_hbm.at[wid], pk_v)
        pltpu.sync_copy(w_hbm.at[pl.ds(wid * EPW_P, EPW_P)], w_v)

        def chunk(c, _):
            for kk in range(K // 16):
                sl = pl.ds(kk * 16, 16)
                v = pk_v[c, sl]
                sbuf[0, sl] = jnp.bitwise_and(v, (1 << SHIFT) - 1)
                dbuf[0, sl] = lax.shift_right_logical(v, SHIFT)

            pltpu.sync_copy(support_hbm.at[sbuf.at[0]], rows_v)

            def scale(kk, _):
                w16 = w_v[pl.ds(c * K + kk * 16, 16)]
                for i in range(16):
                    w = jnp.full((16,), w16[i], jnp.float32)
                    k = kk * 16 + i
                    for j in range(F // 16):
                        sl = pl.ds(j * 16, 16)
                        rows_v[k, sl] = rows_v[k, sl] * w
                return 0
            lax.fori_loop(0, K // 16, scale, 0)

            pltpu.sync_copy(rows_v, acc.at[dbuf.at[0]], add=True)
            return 0
        lax.fori_loop(0, C, chunk, 0)

        plsc.subcore_barrier()
        pltpu.sync_copy(acc.at[pl.ds(base, ROWS_PER_TILE)],
                        out_hbm.at[cid, pl.ds(base, ROWS_PER_TILE)])

    return spmm


_spmm_hid = _make_spmm(NHID)
_spmm_cls = _make_spmm(NCLASS)


_BM = 1000


def _mm1_body(x_ref, w_ref, o_ref):
    o_ref[...] = jnp.dot(x_ref[...], w_ref[...],
                         preferred_element_type=jnp.float32)


def _mm1(x, W1):
    return pl.pallas_call(
        _mm1_body,
        grid=(N // _BM,),
        in_specs=[
            pl.BlockSpec((_BM, NFEAT), lambda i: (i, 0)),
            pl.BlockSpec((NFEAT, NHID), lambda i: (0, 0)),
        ],
        out_specs=pl.BlockSpec((_BM, NHID), lambda i: (i, 0)),
        out_shape=jax.ShapeDtypeStruct((N, NHID), jnp.float32),
    )(x, W1)


def _mid_body(p_ref, b_ref, w_ref, o_ref):
    h = jax.nn.relu(p_ref[0] + p_ref[1] + b_ref[...])
    o_ref[...] = jnp.dot(h, w_ref[...], preferred_element_type=jnp.float32)


def _mid(partials, b1, W2):
    return pl.pallas_call(
        _mid_body,
        grid=(N // _BM,),
        in_specs=[
            pl.BlockSpec((NC, _BM, NHID), lambda i: (0, i, 0)),
            pl.BlockSpec((1, NHID), lambda i: (0, 0)),
            pl.BlockSpec((NHID, NCLASS), lambda i: (0, 0)),
        ],
        out_specs=pl.BlockSpec((_BM, NCLASS), lambda i: (i, 0)),
        out_shape=jax.ShapeDtypeStruct((N, NCLASS), jnp.float32),
    )(partials, b1.reshape(1, NHID), W2)


def _final_body(p_ref, b_ref, o_ref):
    o = p_ref[0] + p_ref[1] + b_ref[...]
    m = jnp.max(o, axis=1, keepdims=True)
    e = jnp.exp(o - m)
    s = jnp.sum(e, axis=1, keepdims=True)
    o_ref[...] = o - m - jnp.log(s)


def _final(partials, b2):
    return pl.pallas_call(
        _final_body,
        grid=(N // _BM,),
        in_specs=[
            pl.BlockSpec((NC, _BM, NCLASS), lambda i: (0, i, 0)),
            pl.BlockSpec((1, NCLASS), lambda i: (0, 0)),
        ],
        out_specs=pl.BlockSpec((_BM, NCLASS), lambda i: (i, 0)),
        out_shape=jax.ShapeDtypeStruct((N, NCLASS), jnp.float32),
    )(partials, b2.reshape(1, NCLASS))


def kernel(x, edge_index, edge_weight, W1, b1, W2, b2):
    pad = EPW_P - EPW
    packed = jnp.pad((edge_index[0] + (edge_index[1] << SHIFT))
                     .reshape(NW, EPW), ((0, 0), (0, pad))).reshape(NW, C, K)
    wpad = jnp.pad(edge_weight.reshape(NW, EPW),
                   ((0, 0), (0, pad))).reshape(NW * EPW_P)

    support1 = _mm1(x, W1)
    p1 = _spmm_hid(support1, packed, wpad)
    support2 = _mid(p1, b1, W2)
    p2 = _spmm_cls(support2, packed, wpad)
    return _final(p2, b2)


# speedup vs baseline: 1.7315x; 1.7315x over previous
import functools

import jax
import jax.numpy as jnp
from jax import lax
from jax.experimental import pallas as pl
from jax.experimental.pallas import tpu as pltpu
from jax.experimental.pallas import tpu_sc as plsc

N = 10000
E = 320000
NFEAT = 128
NHID = 128
NCLASS = 64

NC = 2
NS = 16
NW = NC * NS
EPW = E // NW
K = 80
C = EPW // K
N_PAD = 10240
ROWS_PER_TILE = N_PAD // NS
ZR = 8


def _make_spmm(F):
    mesh = plsc.VectorSubcoreMesh(core_axis_name="c", subcore_axis_name="s")

    @functools.partial(
        pl.kernel,
        mesh=mesh,
        out_type=jax.ShapeDtypeStruct((NC, N_PAD, F), jnp.float32),
        compiler_params=pltpu.CompilerParams(use_tc_tiling_on_sc=False),
        scratch_types=[
            pltpu.VMEM_SHARED((N_PAD, F), jnp.float32),
            pltpu.VMEM((EPW,), jnp.int32),
            pltpu.VMEM((C, K), jnp.int32),
            pltpu.VMEM((EPW,), jnp.float32),
            pltpu.VMEM((K, F), jnp.float32),
            pltpu.VMEM((ZR, F), jnp.float32),
            pltpu.SemaphoreType.DMA,
        ],
    )
    def spmm(support_hbm, src_hbm, dst_hbm, w_hbm, out_hbm,
             acc, src_v, dst_v, w_v, rows_v, zeros_v, zsem):
        cid = lax.axis_index("c")
        sid = lax.axis_index("s")
        wid = sid * NC + cid

        for r in range(ZR):
            for j in range(F // 16):
                zeros_v[r, pl.ds(j * 16, 16)] = jnp.zeros((16,), jnp.float32)
        base = sid * ROWS_PER_TILE
        NZ = ROWS_PER_TILE // ZR

        def zissue(i, _):
            pltpu.async_copy(zeros_v, acc.at[pl.ds(base + i * ZR, ZR)], zsem)
            return 0
        lax.fori_loop(0, NZ, zissue, 0)

        def zdrain(i, _):
            pltpu.make_async_copy(zeros_v, acc.at[pl.ds(base, ZR)], zsem).wait()
            return 0
        lax.fori_loop(0, NZ, zdrain, 0)
        plsc.subcore_barrier()

        pltpu.sync_copy(src_hbm.at[pl.ds(wid * EPW, EPW)], src_v)
        pltpu.sync_copy(dst_hbm.at[wid], dst_v)
        pltpu.sync_copy(w_hbm.at[pl.ds(wid * EPW, EPW)], w_v)

        def chunk(c, _):
            pltpu.sync_copy(support_hbm.at[src_v.at[pl.ds(c * K, K)]], rows_v)

            def scale(kk, _):
                w16 = w_v[pl.ds(c * K + kk * 16, 16)]
                for i in range(16):
                    w = jnp.full((16,), w16[i], jnp.float32)
                    k = kk * 16 + i
                    for j in range(F // 16):
                        sl = pl.ds(j * 16, 16)
                        rows_v[k, sl] = rows_v[k, sl] * w
                return 0
            lax.fori_loop(0, K // 16, scale, 0)

            pltpu.sync_copy(rows_v, acc.at[dst_v.at[c]], add=True)
            return 0
        lax.fori_loop(0, C, chunk, 0)

        plsc.subcore_barrier()
        pltpu.sync_copy(acc.at[pl.ds(base, ROWS_PER_TILE)],
                        out_hbm.at[cid, pl.ds(base, ROWS_PER_TILE)])

    return spmm


_spmm_hid = _make_spmm(NHID)
_spmm_cls = _make_spmm(NCLASS)


_BM = 1000


def _mm1_body(x_ref, w_ref, o_ref):
    o_ref[...] = jnp.dot(x_ref[...], w_ref[...],
                         preferred_element_type=jnp.float32)


def _mm1(x, W1):
    return pl.pallas_call(
        _mm1_body,
        grid=(N // _BM,),
        in_specs=[
            pl.BlockSpec((_BM, NFEAT), lambda i: (i, 0)),
            pl.BlockSpec((NFEAT, NHID), lambda i: (0, 0)),
        ],
        out_specs=pl.BlockSpec((_BM, NHID), lambda i: (i, 0)),
        out_shape=jax.ShapeDtypeStruct((N, NHID), jnp.float32),
    )(x, W1)


def _mid_body(p_ref, b_ref, w_ref, o_ref):
    h = jax.nn.relu(p_ref[0] + p_ref[1] + b_ref[...])
    o_ref[...] = jnp.dot(h, w_ref[...], preferred_element_type=jnp.float32)


def _mid(partials, b1, W2):
    return pl.pallas_call(
        _mid_body,
        grid=(N // _BM,),
        in_specs=[
            pl.BlockSpec((NC, _BM, NHID), lambda i: (0, i, 0)),
            pl.BlockSpec((1, NHID), lambda i: (0, 0)),
            pl.BlockSpec((NHID, NCLASS), lambda i: (0, 0)),
        ],
        out_specs=pl.BlockSpec((_BM, NCLASS), lambda i: (i, 0)),
        out_shape=jax.ShapeDtypeStruct((N, NCLASS), jnp.float32),
    )(partials, b1.reshape(1, NHID), W2)


def _final_body(p_ref, b_ref, o_ref):
    o = p_ref[0] + p_ref[1] + b_ref[...]
    m = jnp.max(o, axis=1, keepdims=True)
    e = jnp.exp(o - m)
    s = jnp.sum(e, axis=1, keepdims=True)
    o_ref[...] = o - m - jnp.log(s)


def _final(partials, b2):
    return pl.pallas_call(
        _final_body,
        grid=(N // _BM,),
        in_specs=[
            pl.BlockSpec((NC, _BM, NCLASS), lambda i: (0, i, 0)),
            pl.BlockSpec((1, NCLASS), lambda i: (0, 0)),
        ],
        out_specs=pl.BlockSpec((_BM, NCLASS), lambda i: (i, 0)),
        out_shape=jax.ShapeDtypeStruct((N, NCLASS), jnp.float32),
    )(partials, b2.reshape(1, NCLASS))


def kernel(x, edge_index, edge_weight, W1, b1, W2, b2):
    src = edge_index[0]
    dst = edge_index[1].reshape(NW, C, K)
    w = edge_weight

    support1 = _mm1(x, W1)
    p1 = _spmm_hid(support1, src, dst, w)
    support2 = _mid(p1, b1, W2)
    p2 = _spmm_cls(support2, src, dst, w)
    return _final(p2, b2)


# speedup vs baseline: 2.1991x; 1.2701x over previous
import functools

import jax
import jax.numpy as jnp
from jax import lax
from jax.experimental import pallas as pl
from jax.experimental.pallas import tpu as pltpu
from jax.experimental.pallas import tpu_sc as plsc

N = 10000
E = 320000
NFEAT = 128
NHID = 128
NCLASS = 64

NC = 2
NS = 16
NW = NC * NS
EPW = E // NW
K = 80
C = EPW // K
N_PAD = 10240
ROWS_PER_TILE = N_PAD // NS
ZR = 8


def _make_spmm(F):
    mesh = plsc.VectorSubcoreMesh(core_axis_name="c", subcore_axis_name="s")

    @functools.partial(
        pl.kernel,
        mesh=mesh,
        out_type=jax.ShapeDtypeStruct((NC, N_PAD, F), jnp.float32),
        compiler_params=pltpu.CompilerParams(use_tc_tiling_on_sc=False),
        scratch_types=[
            pltpu.VMEM_SHARED((N_PAD, F), jnp.float32),
            pltpu.VMEM((EPW,), jnp.int32),
            pltpu.VMEM((C, K), jnp.int32),
            pltpu.VMEM((EPW,), jnp.float32),
            pltpu.VMEM((K, F), jnp.float32),
            pltpu.VMEM((ZR, F), jnp.float32),
            pltpu.SemaphoreType.DMA,
        ],
    )
    def spmm(support_hbm, src_hbm, dst_hbm, w_hbm, out_hbm,
             acc, src_v, dst_v, w_v, rows_v, zeros_v, zsem):
        cid = lax.axis_index("c")
        sid = lax.axis_index("s")
        wid = sid * NC + cid

        for r in range(ZR):
            for j in range(F // 16):
                zeros_v[r, pl.ds(j * 16, 16)] = jnp.zeros((16,), jnp.float32)
        base = sid * ROWS_PER_TILE
        NZ = ROWS_PER_TILE // ZR

        def zissue(i, _):
            pltpu.async_copy(zeros_v, acc.at[pl.ds(base + i * ZR, ZR)], zsem)
            return 0
        lax.fori_loop(0, NZ, zissue, 0)

        def zdrain(i, _):
            pltpu.make_async_copy(zeros_v, acc.at[pl.ds(base, ZR)], zsem).wait()
            return 0
        lax.fori_loop(0, NZ, zdrain, 0)
        plsc.subcore_barrier()

        pltpu.sync_copy(src_hbm.at[pl.ds(wid * EPW, EPW)], src_v)
        pltpu.sync_copy(dst_hbm.at[wid], dst_v)
        pltpu.sync_copy(w_hbm.at[pl.ds(wid * EPW, EPW)], w_v)

        def chunk(c, _):
            pltpu.sync_copy(support_hbm.at[src_v.at[pl.ds(c * K, K)]], rows_v)

            @plsc.parallel_loop(0, K // 16, unroll=2)
            def scale(kk):
                w16 = w_v[pl.ds(c * K + kk * 16, 16)]
                for i in range(16):
                    w = jnp.full((16,), w16[i], jnp.float32)
                    k = kk * 16 + i
                    for j in range(F // 16):
                        sl = pl.ds(j * 16, 16)
                        rows_v[k, sl] = rows_v[k, sl] * w

            pltpu.sync_copy(rows_v, acc.at[dst_v.at[c]], add=True)
            return 0
        lax.fori_loop(0, C, chunk, 0)

        plsc.subcore_barrier()
        pltpu.sync_copy(acc.at[pl.ds(base, ROWS_PER_TILE)],
                        out_hbm.at[cid, pl.ds(base, ROWS_PER_TILE)])

    return spmm


_spmm_hid = _make_spmm(NHID)
_spmm_cls = _make_spmm(NCLASS)


_BM = 1000


def _mm1_body(x_ref, w_ref, o_ref):
    o_ref[...] = jnp.dot(x_ref[...], w_ref[...],
                         preferred_element_type=jnp.float32)


def _mm1(x, W1):
    return pl.pallas_call(
        _mm1_body,
        grid=(N // _BM,),
        in_specs=[
            pl.BlockSpec((_BM, NFEAT), lambda i: (i, 0)),
            pl.BlockSpec((NFEAT, NHID), lambda i: (0, 0)),
        ],
        out_specs=pl.BlockSpec((_BM, NHID), lambda i: (i, 0)),
        out_shape=jax.ShapeDtypeStruct((N, NHID), jnp.float32),
    )(x, W1)


def _mid_body(p_ref, b_ref, w_ref, o_ref):
    h = jax.nn.relu(p_ref[0] + p_ref[1] + b_ref[...])
    o_ref[...] = jnp.dot(h, w_ref[...], preferred_element_type=jnp.float32)


def _mid(partials, b1, W2):
    return pl.pallas_call(
        _mid_body,
        grid=(N // _BM,),
        in_specs=[
            pl.BlockSpec((NC, _BM, NHID), lambda i: (0, i, 0)),
            pl.BlockSpec((1, NHID), lambda i: (0, 0)),
            pl.BlockSpec((NHID, NCLASS), lambda i: (0, 0)),
        ],
        out_specs=pl.BlockSpec((_BM, NCLASS), lambda i: (i, 0)),
        out_shape=jax.ShapeDtypeStruct((N, NCLASS), jnp.float32),
    )(partials, b1.reshape(1, NHID), W2)


def _final_body(p_ref, b_ref, o_ref):
    o = p_ref[0] + p_ref[1] + b_ref[...]
    m = jnp.max(o, axis=1, keepdims=True)
    e = jnp.exp(o - m)
    s = jnp.sum(e, axis=1, keepdims=True)
    o_ref[...] = o - m - jnp.log(s)


def _final(partials, b2):
    return pl.pallas_call(
        _final_body,
        grid=(N // _BM,),
        in_specs=[
            pl.BlockSpec((NC, _BM, NCLASS), lambda i: (0, i, 0)),
            pl.BlockSpec((1, NCLASS), lambda i: (0, 0)),
        ],
        out_specs=pl.BlockSpec((_BM, NCLASS), lambda i: (i, 0)),
        out_shape=jax.ShapeDtypeStruct((N, NCLASS), jnp.float32),
    )(partials, b2.reshape(1, NCLASS))


def kernel(x, edge_index, edge_weight, W1, b1, W2, b2):
    src = edge_index[0]
    dst = edge_index[1].reshape(NW, C, K)
    w = edge_weight

    support1 = _mm1(x, W1)
    p1 = _spmm_hid(support1, src, dst, w)
    support2 = _mid(p1, b1, W2)
    p2 = _spmm_cls(support2, src, dst, w)
    return _final(p2, b2)


# speedup vs baseline: 2.2098x; 1.0049x over previous
import functools

import jax
import jax.numpy as jnp
from jax import lax
from jax.experimental import pallas as pl
from jax.experimental.pallas import tpu as pltpu
from jax.experimental.pallas import tpu_sc as plsc

N = 10000
E = 320000
NFEAT = 128
NHID = 128
NCLASS = 64

NC = 2
NS = 16
NW = NC * NS
EPW = E // NW
K = 80
C = EPW // K
N_PAD = 10240
ROWS_PER_TILE = N_PAD // NS
ZR = 8


def _make_spmm(F):
    mesh = plsc.VectorSubcoreMesh(core_axis_name="c", subcore_axis_name="s")

    @functools.partial(
        pl.kernel,
        mesh=mesh,
        out_type=jax.ShapeDtypeStruct((NC, N_PAD, F), jnp.float32),
        compiler_params=pltpu.CompilerParams(use_tc_tiling_on_sc=False),
        scratch_types=[
            pltpu.VMEM_SHARED((N_PAD, F), jnp.float32),
            pltpu.VMEM((EPW,), jnp.int32),
            pltpu.VMEM((C, K), jnp.int32),
            pltpu.VMEM((EPW,), jnp.float32),
            pltpu.VMEM((K, F), jnp.float32),
            pltpu.VMEM((ZR, F), jnp.float32),
            pltpu.SemaphoreType.DMA,
        ],
    )
    def spmm(support_hbm, src_hbm, dst_hbm, w_hbm, out_hbm,
             acc, src_v, dst_v, w_v, rows_v, zeros_v, zsem):
        cid = lax.axis_index("c")
        sid = lax.axis_index("s")
        wid = sid * NC + cid

        for r in range(ZR):
            for j in range(F // 16):
                zeros_v[r, pl.ds(j * 16, 16)] = jnp.zeros((16,), jnp.float32)
        base = sid * ROWS_PER_TILE
        NZ = ROWS_PER_TILE // ZR

        def zissue(i, _):
            pltpu.async_copy(zeros_v, acc.at[pl.ds(base + i * ZR, ZR)], zsem)
            return 0
        lax.fori_loop(0, NZ, zissue, 0)

        def zdrain(i, _):
            pltpu.make_async_copy(zeros_v, acc.at[pl.ds(base, ZR)], zsem).wait()
            return 0
        lax.fori_loop(0, NZ, zdrain, 0)
        plsc.subcore_barrier()

        pltpu.sync_copy(src_hbm.at[pl.ds(wid * EPW, EPW)], src_v)
        pltpu.sync_copy(dst_hbm.at[wid], dst_v)
        pltpu.sync_copy(w_hbm.at[pl.ds(wid * EPW, EPW)], w_v)

        def chunk(c, _):
            pltpu.sync_copy(support_hbm.at[src_v.at[pl.ds(c * K, K)]], rows_v)

            @plsc.parallel_loop(0, K // 16, unroll=5)
            def scale(kk):
                w16 = w_v[pl.ds(c * K + kk * 16, 16)]
                for i in range(16):
                    w = jnp.full((16,), w16[i], jnp.float32)
                    k = kk * 16 + i
                    for j in range(F // 16):
                        sl = pl.ds(j * 16, 16)
                        rows_v[k, sl] = rows_v[k, sl] * w

            pltpu.sync_copy(rows_v, acc.at[dst_v.at[c]], add=True)
            return 0
        lax.fori_loop(0, C, chunk, 0)

        plsc.subcore_barrier()
        pltpu.sync_copy(acc.at[pl.ds(base, ROWS_PER_TILE)],
                        out_hbm.at[cid, pl.ds(base, ROWS_PER_TILE)])

    return spmm


_spmm_hid = _make_spmm(NHID)
_spmm_cls = _make_spmm(NCLASS)


_BM = 1000


def _mm1_body(x_ref, w_ref, o_ref):
    o_ref[...] = jnp.dot(x_ref[...], w_ref[...],
                         preferred_element_type=jnp.float32)


def _mm1(x, W1):
    return pl.pallas_call(
        _mm1_body,
        grid=(N // _BM,),
        in_specs=[
            pl.BlockSpec((_BM, NFEAT), lambda i: (i, 0)),
            pl.BlockSpec((NFEAT, NHID), lambda i: (0, 0)),
        ],
        out_specs=pl.BlockSpec((_BM, NHID), lambda i: (i, 0)),
        out_shape=jax.ShapeDtypeStruct((N, NHID), jnp.float32),
    )(x, W1)


def _mid_body(p_ref, b_ref, w_ref, o_ref):
    h = jax.nn.relu(p_ref[0] + p_ref[1] + b_ref[...])
    o_ref[...] = jnp.dot(h, w_ref[...], preferred_element_type=jnp.float32)


def _mid(partials, b1, W2):
    return pl.pallas_call(
        _mid_body,
        grid=(N // _BM,),
        in_specs=[
            pl.BlockSpec((NC, _BM, NHID), lambda i: (0, i, 0)),
            pl.BlockSpec((1, NHID), lambda i: (0, 0)),
            pl.BlockSpec((NHID, NCLASS), lambda i: (0, 0)),
        ],
        out_specs=pl.BlockSpec((_BM, NCLASS), lambda i: (i, 0)),
        out_shape=jax.ShapeDtypeStruct((N, NCLASS), jnp.float32),
    )(partials, b1.reshape(1, NHID), W2)


def _final_body(p_ref, b_ref, o_ref):
    o = p_ref[0] + p_ref[1] + b_ref[...]
    m = jnp.max(o, axis=1, keepdims=True)
    e = jnp.exp(o - m)
    s = jnp.sum(e, axis=1, keepdims=True)
    o_ref[...] = o - m - jnp.log(s)


def _final(partials, b2):
    return pl.pallas_call(
        _final_body,
        grid=(N // _BM,),
        in_specs=[
            pl.BlockSpec((NC, _BM, NCLASS), lambda i: (0, i, 0)),
            pl.BlockSpec((1, NCLASS), lambda i: (0, 0)),
        ],
        out_specs=pl.BlockSpec((_BM, NCLASS), lambda i: (i, 0)),
        out_shape=jax.ShapeDtypeStruct((N, NCLASS), jnp.float32),
    )(partials, b2.reshape(1, NCLASS))


def kernel(x, edge_index, edge_weight, W1, b1, W2, b2):
    src = edge_index[0]
    dst = edge_index[1].reshape(NW, C, K)
    w = edge_weight

    support1 = _mm1(x, W1)
    p1 = _spmm_hid(support1, src, dst, w)
    support2 = _mid(p1, b1, W2)
    p2 = _spmm_cls(support2, src, dst, w)
    return _final(p2, b2)


# speedup vs baseline: 2.4617x; 1.1140x over previous
import functools

import jax
import jax.numpy as jnp
from jax import lax
from jax.experimental import pallas as pl
from jax.experimental.pallas import tpu as pltpu
from jax.experimental.pallas import tpu_sc as plsc

N = 10000
E = 320000
NFEAT = 128
NHID = 128
NCLASS = 64

NC = 2
NS = 16
NW = NC * NS
EPW = E // NW
K = 80
C = EPW // K
N_PAD = 10240
ROWS_PER_TILE = N_PAD // NS
ZR = 8


def _make_spmm(F, stage_support=False):
    mesh = plsc.VectorSubcoreMesh(core_axis_name="c", subcore_axis_name="s")

    @functools.partial(
        pl.kernel,
        mesh=mesh,
        out_type=jax.ShapeDtypeStruct((NC, N_PAD, F), jnp.float32),
        compiler_params=pltpu.CompilerParams(use_tc_tiling_on_sc=False),
        scratch_types=(
            ([pltpu.VMEM_SHARED((N_PAD, F), jnp.float32)] if stage_support
             else []) + [
            pltpu.VMEM_SHARED((N_PAD, F), jnp.float32),
            pltpu.VMEM((EPW,), jnp.int32),
            pltpu.VMEM((C, K), jnp.int32),
            pltpu.VMEM((EPW,), jnp.float32),
            pltpu.VMEM((K, F), jnp.float32),
            pltpu.VMEM((ZR, F), jnp.float32),
            pltpu.SemaphoreType.DMA,
        ]),
    )
    def spmm(support_hbm, src_hbm, dst_hbm, w_hbm, out_hbm, *refs):
        if stage_support:
            sup_sh = refs[0]
            refs = refs[1:]
        acc, src_v, dst_v, w_v, rows_v, zeros_v, zsem = refs
        cid = lax.axis_index("c")
        sid = lax.axis_index("s")
        wid = sid * NC + cid

        for r in range(ZR):
            for j in range(F // 16):
                zeros_v[r, pl.ds(j * 16, 16)] = jnp.zeros((16,), jnp.float32)
        base = sid * ROWS_PER_TILE
        NZ = ROWS_PER_TILE // ZR

        def zissue(i, _):
            pltpu.async_copy(zeros_v, acc.at[pl.ds(base + i * ZR, ZR)], zsem)
            return 0
        lax.fori_loop(0, NZ, zissue, 0)

        def zdrain(i, _):
            pltpu.make_async_copy(zeros_v, acc.at[pl.ds(base, ZR)], zsem).wait()
            return 0
        lax.fori_loop(0, NZ, zdrain, 0)
        if stage_support:
            pltpu.sync_copy(support_hbm.at[pl.ds(base, ROWS_PER_TILE)],
                            sup_sh.at[pl.ds(base, ROWS_PER_TILE)])
        plsc.subcore_barrier()

        pltpu.sync_copy(src_hbm.at[pl.ds(wid * EPW, EPW)], src_v)
        pltpu.sync_copy(dst_hbm.at[wid], dst_v)
        pltpu.sync_copy(w_hbm.at[pl.ds(wid * EPW, EPW)], w_v)

        sup = sup_sh if stage_support else support_hbm

        def chunk(c, _):
            pltpu.sync_copy(sup.at[src_v.at[pl.ds(c * K, K)]], rows_v)

            @plsc.parallel_loop(0, K // 16, unroll=5)
            def scale(kk):
                w16 = w_v[pl.ds(c * K + kk * 16, 16)]
                for i in range(16):
                    w = jnp.full((16,), w16[i], jnp.float32)
                    k = kk * 16 + i
                    for j in range(F // 16):
                        sl = pl.ds(j * 16, 16)
                        rows_v[k, sl] = rows_v[k, sl] * w

            pltpu.sync_copy(rows_v, acc.at[dst_v.at[c]], add=True)
            return 0
        lax.fori_loop(0, C, chunk, 0)

        plsc.subcore_barrier()
        pltpu.sync_copy(acc.at[pl.ds(base, ROWS_PER_TILE)],
                        out_hbm.at[cid, pl.ds(base, ROWS_PER_TILE)])

    return spmm


_spmm_hid = _make_spmm(NHID)
_spmm_cls = _make_spmm(NCLASS, stage_support=True)


_BM = 1000


def _mm1_body(x_ref, w_ref, o_ref):
    o_ref[...] = jnp.dot(x_ref[...], w_ref[...],
                         preferred_element_type=jnp.float32)


def _mm1(x, W1):
    return pl.pallas_call(
        _mm1_body,
        grid=(N // _BM,),
        in_specs=[
            pl.BlockSpec((_BM, NFEAT), lambda i: (i, 0)),
            pl.BlockSpec((NFEAT, NHID), lambda i: (0, 0)),
        ],
        out_specs=pl.BlockSpec((_BM, NHID), lambda i: (i, 0)),
        out_shape=jax.ShapeDtypeStruct((N, NHID), jnp.float32),
    )(x, W1)


def _mid_body(p_ref, b_ref, w_ref, o_ref):
    h = jax.nn.relu(p_ref[0] + p_ref[1] + b_ref[...])
    o_ref[...] = jnp.dot(h, w_ref[...], preferred_element_type=jnp.float32)


_BM2 = 1024


def _mid(partials, b1, W2):
    return pl.pallas_call(
        _mid_body,
        grid=(N_PAD // _BM2,),
        in_specs=[
            pl.BlockSpec((NC, _BM2, NHID), lambda i: (0, i, 0)),
            pl.BlockSpec((1, NHID), lambda i: (0, 0)),
            pl.BlockSpec((NHID, NCLASS), lambda i: (0, 0)),
        ],
        out_specs=pl.BlockSpec((_BM2, NCLASS), lambda i: (i, 0)),
        out_shape=jax.ShapeDtypeStruct((N_PAD, NCLASS), jnp.float32),
    )(partials, b1.reshape(1, NHID), W2)


def _final_body(p_ref, b_ref, o_ref):
    o = p_ref[0] + p_ref[1] + b_ref[...]
    m = jnp.max(o, axis=1, keepdims=True)
    e = jnp.exp(o - m)
    s = jnp.sum(e, axis=1, keepdims=True)
    o_ref[...] = o - m - jnp.log(s)


def _final(partials, b2):
    return pl.pallas_call(
        _final_body,
        grid=(N // _BM,),
        in_specs=[
            pl.BlockSpec((NC, _BM, NCLASS), lambda i: (0, i, 0)),
            pl.BlockSpec((1, NCLASS), lambda i: (0, 0)),
        ],
        out_specs=pl.BlockSpec((_BM, NCLASS), lambda i: (i, 0)),
        out_shape=jax.ShapeDtypeStruct((N, NCLASS), jnp.float32),
    )(partials, b2.reshape(1, NCLASS))


def kernel(x, edge_index, edge_weight, W1, b1, W2, b2):
    src = edge_index[0]
    dst = edge_index[1].reshape(NW, C, K)
    w = edge_weight

    support1 = _mm1(x, W1)
    p1 = _spmm_hid(support1, src, dst, w)
    support2 = _mid(p1, b1, W2)
    p2 = _spmm_cls(support2, src, dst, w)
    return _final(p2, b2)
